# baseline (speedup 1.0000x reference)
import jax
import jax.numpy as jnp
from jax import lax
from jax.experimental import pallas as pl
from jax.experimental.pallas import tpu as pltpu
from jax.experimental.pallas import tpu_sc as plsc

N = 10000
H = 256
K = 16
C = 64
NC = 2
NS = 16
CH = 128
NPAD = 10240


def _leaky(x):
    return jnp.where(x >= 0, x, 0.01 * x)


GB = 8
NB = 3


def _make_segsum(EPAD, colsplit):
    ncores_split = 1 if colsplit else NC
    tchunks = EPAD // (NS * ncores_split * CH)
    ngroups = tchunks // GB
    srows = EPAD // CH
    zrows = NPAD // NS
    mesh = plsc.VectorSubcoreMesh(
        core_axis_name="c", subcore_axis_name="s", num_cores=NC, num_subcores=NS
    )

    def body(feat, src2, dst2, zeros, out,
             sa, sb, da, db, r0, r1, acc, gsem, isem):
        rbufs = (r0, r1)
        c = lax.axis_index("c")
        s = lax.axis_index("s")
        z0 = s * zrows
        pltpu.sync_copy(zeros.at[pl.ds(z0, zrows)], acc.at[pl.ds(z0, zrows)])
        plsc.subcore_barrier()
        if colsplit:
            crow = c * srows
            rbase = s * tchunks
        else:
            crow = 0
            rbase = (c * NS + s) * tchunks

        def fire_idx(blk, sdst, ddst):
            row0 = rbase + blk * GB
            pltpu.async_copy(src2.at[pl.ds(crow + row0, GB)], sdst, isem)
            pltpu.async_copy(dst2.at[pl.ds(row0, GB)], ddst, isem)

        def wait_idx(sdst, ddst):
            pltpu.make_async_copy(src2.at[pl.ds(0, GB)], sdst, isem).wait()
            pltpu.make_async_copy(dst2.at[pl.ds(0, GB)], ddst, isem).wait()

        def group(g, cs, cd, ns_, nd_):
            for q in range(GB):
                b = q % 2
                pltpu.make_async_copy(
                    zeros.at[pl.ds(0, CH)], rbufs[b], gsem.at[b]).wait()
                if q == GB - 1:
                    wait_idx(ns_, nd_)
                    nidx = ns_.at[0]
                else:
                    nidx = cs.at[q + 1]
                pltpu.async_copy(feat.at[nidx], rbufs[1 - b], gsem.at[1 - b])
                pltpu.sync_copy(rbufs[b], acc.at[cd.at[q]], add=True)
            fire_idx(g + 2, cs, cd)

        pltpu.sync_copy(src2.at[pl.ds(crow + rbase, GB)], sa)
        pltpu.sync_copy(dst2.at[pl.ds(rbase, GB)], da)
        fire_idx(1, sb, db)
        pltpu.async_copy(feat.at[sa.at[0]], r0, gsem.at[0])

        def two_groups(gg, carry):
            group(2 * gg, sa, da, sb, db)
            group(2 * gg + 1, sb, db, sa, da)
            return carry

        lax.fori_loop(0, ngroups // 2, two_groups, 0)
        pltpu.make_async_copy(zeros.at[pl.ds(0, CH)], r0, gsem.at[0]).wait()
        wait_idx(sa, da)
        plsc.subcore_barrier()
        w0 = s * zrows
        pltpu.sync_copy(acc.at[pl.ds(w0, zrows)], out.at[pl.ds(c * NPAD + w0, zrows)])

    return pl.kernel(
        body,
        out_type=jax.ShapeDtypeStruct((2 * NPAD, 128), jnp.float32),
        mesh=mesh,
        scratch_types=[
            pltpu.VMEM((GB, CH), jnp.int32),
            pltpu.VMEM((GB, CH), jnp.int32),
            pltpu.VMEM((GB, CH), jnp.int32),
            pltpu.VMEM((GB, CH), jnp.int32),
            pltpu.VMEM((CH, 128), jnp.float32),
            pltpu.VMEM((CH, 128), jnp.float32),
            pltpu.VMEM_SHARED((NPAD, 128), jnp.float32),
            pltpu.SemaphoreType.DMA((2,)),
            pltpu.SemaphoreType.DMA,
        ],
    )


def _make_gin(Df, agg_mode, B):
    G = N // B

    def body(fl, fr, al, ar, w, b, sc, a_mat, h3, pt, lout, m):
        i = pl.program_id(0)
        f = jnp.concatenate([fl[0], fr[0]], axis=1)
        if agg_mode == "concat":
            agg = jnp.concatenate([al[0], ar[0]], axis=1)
        else:
            agg = al[0] + ar[0]
        t = sc[0, 0] * f + agg
        h = jnp.dot(t, w[...], preferred_element_type=jnp.float32) + b[...]
        h = _leaky(h)
        h3[0] = h[:, : H // 2]
        h3[1] = h[:, H // 2 :]
        s = jnp.dot(h, a_mat[...], preferred_element_type=jnp.float32)
        bm = jnp.max(s, axis=0, keepdims=True)

        @pl.when(i == 0)
        def _():
            p = jnp.exp(s - bm)
            m[...] = bm
            lout[...] = jnp.sum(p, axis=0, keepdims=True)
            pt[...] = lax.dot_general(
                h, p, (((0,), (0,)), ((), ())), preferred_element_type=jnp.float32
            )

        @pl.when(i > 0)
        def _():
            mn = jnp.maximum(m[...], bm)
            alpha = jnp.exp(m[...] - mn)
            p = jnp.exp(s - mn)
            m[...] = mn
            lout[...] = lout[...] * alpha + jnp.sum(p, axis=0, keepdims=True)
            pt[...] = pt[...] * alpha + lax.dot_general(
                h, p, (((0,), (0,)), ((), ())), preferred_element_type=jnp.float32
            )

    D = 2 * Df
    wa = 128 if agg_mode == "sum" else 128
    return pl.pallas_call(
        body,
        grid=(G,),
        in_specs=[
            pl.BlockSpec((1, B, Df), lambda i: (0, i, 0)),
            pl.BlockSpec((1, B, Df), lambda i: (1, i, 0)),
            pl.BlockSpec((1, B, wa), lambda i: (0, i, 0)),
            pl.BlockSpec((1, B, wa), lambda i: (1, i, 0)),
            pl.BlockSpec((D, H), lambda i: (0, 0)),
            pl.BlockSpec((1, H), lambda i: (0, 0)),
            pl.BlockSpec((1, 1), lambda i: (0, 0), memory_space=pltpu.SMEM),
            pl.BlockSpec((H, K), lambda i: (0, 0)),
        ],
        out_specs=[
            pl.BlockSpec((2, B, H // 2), lambda i: (0, i, 0)),
            pl.BlockSpec((H, K), lambda i: (0, 0)),
            pl.BlockSpec((1, K), lambda i: (0, 0)),
        ],
        out_shape=[
            jax.ShapeDtypeStruct((2, N, H // 2), jnp.float32),
            jax.ShapeDtypeStruct((H, K), jnp.float32),
            jax.ShapeDtypeStruct((1, K), jnp.float32),
        ],
        scratch_shapes=[pltpu.VMEM((1, K), jnp.float32)],
    )


def _head1_body(p0, l0, p1, l1, p2, l2, wc, bc, out):
    merged_t = jnp.concatenate(
        [p0[...] / l0[...], p1[...] / l1[...], p2[...] / l2[...]], axis=0
    )
    conv_t = _leaky(
        jnp.dot(wc[...], merged_t, preferred_element_type=jnp.float32) + bc[...]
    )
    out[...] = conv_t


_head1 = pl.pallas_call(
    _head1_body, out_shape=jax.ShapeDtypeStruct((C, K), jnp.float32)
)


def _head2_body(fc, wd1, bd1, wd2, bd2, out):
    d1 = _leaky(
        lax.dot_general(
            fc[...], wd1[...], (((1,), (1,)), ((), ())),
            preferred_element_type=jnp.float32,
        )
        + bd1[...]
    )
    o = (
        lax.dot_general(
            d1, wd2[...], (((1,), (1,)), ((), ())),
            preferred_element_type=jnp.float32,
        )
        + bd2[...]
    )
    out[...] = 1.0 / (1.0 + jnp.exp(-o))


_head2 = pl.pallas_call(
    _head2_body, out_shape=jax.ShapeDtypeStruct((1, 2), jnp.float32)
)

_SEGSUM = {}
_GIN = {}


def _segsum(feat, src2, dst2, zeros, epad, colsplit):
    key = (epad, colsplit)
    if key not in _SEGSUM:
        _SEGSUM[key] = _make_segsum(epad, colsplit)
    return _SEGSUM[key](feat, src2, dst2, zeros)


def _gin(f3, agg3, w, b, scale, a_mat, df, agg_mode):
    key = (df, agg_mode)
    if key not in _GIN:
        _GIN[key] = _make_gin(df, agg_mode, 1000)
    return _GIN[key](f3, f3, agg3, agg3, w, b, scale, a_mat)


def kernel(x, edge_index, W0, b0, eps0, A0, W1, b1, eps1, A1, W2, b2, eps2, A2,
           Wc, bc, Wd1, bd1, Wd2, bd2):
    src = edge_index[0]
    dst = edge_index[1]
    e = src.shape[0]
    step = NC * NS * CH * 2 * GB
    epad = -(-e // step) * step
    pad = epad - e
    srcp = jnp.concatenate([src, jnp.zeros((pad,), jnp.int32)])
    dstp = jnp.concatenate([dst, jnp.full((pad,), N, jnp.int32)])
    lookahead = jnp.full((2 * GB * CH,), 0, jnp.int32)
    src2 = jnp.concatenate([srcp, srcp + N, lookahead]).reshape(-1, CH)
    dst2 = jnp.concatenate(
        [dstp, jnp.full((2 * GB * CH,), N, jnp.int32)]).reshape(-1, CH)
    zeros128 = jnp.zeros((NPAD, 128), jnp.float32)

    x3 = x.reshape(N, 2, 64).transpose(1, 0, 2)
    agg0 = _segsum(x, src2, dst2, zeros128, epad, False)
    h0, p0, l0 = _gin(x3, agg0.reshape(2, NPAD, 128), W0, b0.reshape(1, H),
                      (1.0 + eps0).reshape(1, 1), A0, 64, "sum")

    agg1 = _segsum(h0.reshape(2 * N, 128), src2, dst2, zeros128, epad, True)
    h1, p1, l1 = _gin(h0, agg1.reshape(2, NPAD, 128), W1, b1.reshape(1, H),
                      (1.0 + eps1).reshape(1, 1), A1, 128, "concat")

    agg2 = _segsum(h1.reshape(2 * N, 128), src2, dst2, zeros128, epad, True)
    h2, p2, l2 = _gin(h1, agg2.reshape(2, NPAD, 128), W2, b2.reshape(1, H),
                      (1.0 + eps2).reshape(1, 1), A2, 128, "concat")

    conv_t = _head1(p0, l0, p1, l1, p2, l2, Wc, bc.reshape(C, 1))
    fc = conv_t.reshape(1, C * K)
    return _head2(fc, Wd1, bd1.reshape(1, 128), Wd2, bd2.reshape(1, 2))


# speedup vs baseline: 1.1261x; 1.1261x over previous
import jax
import jax.numpy as jnp
from jax import lax
from jax.experimental import pallas as pl
from jax.experimental.pallas import tpu as pltpu
from jax.experimental.pallas import tpu_sc as plsc

N = 10000
H = 256
K = 16
C = 64
NC = 2
NS = 16
CH = 128
NPAD = 10240


def _leaky(x):
    return jnp.where(x >= 0, x, 0.01 * x)


GB = 8
NB = 3


def _make_segsum(EPAD, colsplit):
    ncores_split = 1 if colsplit else NC
    tchunks = EPAD // (NS * ncores_split * CH)
    zrows = NPAD // NS
    mesh = plsc.VectorSubcoreMesh(
        core_axis_name="c", subcore_axis_name="s", num_cores=NC, num_subcores=NS
    )

    def body(feat, srcp, dstp, zeros, out, sidx, didx, rows, acc, gsem):
        c = lax.axis_index("c")
        s = lax.axis_index("s")
        z0 = s * zrows
        pltpu.sync_copy(zeros.at[pl.ds(z0, zrows)], acc.at[pl.ds(z0, zrows)])
        plsc.subcore_barrier()
        if colsplit:
            dbase = s * tchunks * CH
            ebase = c * EPAD + dbase
        else:
            dbase = (c * NS + s) * tchunks * CH
            ebase = dbase

        def chunk(j, carry):
            off = j * CH
            pltpu.sync_copy(srcp.at[pl.ds(ebase + off, CH)], sidx)
            pltpu.sync_copy(dstp.at[pl.ds(dbase + off, CH)], didx)
            pltpu.async_copy(feat.at[sidx], rows, gsem).wait()
            pltpu.sync_copy(rows, acc.at[didx], add=True)
            return carry

        lax.fori_loop(0, tchunks, chunk, 0)
        plsc.subcore_barrier()
        w0 = s * zrows
        pltpu.sync_copy(acc.at[pl.ds(w0, zrows)], out.at[pl.ds(c * NPAD + w0, zrows)])

    return pl.kernel(
        body,
        out_type=jax.ShapeDtypeStruct((2 * NPAD, 128), jnp.float32),
        mesh=mesh,
        scratch_types=[
            pltpu.VMEM((CH,), jnp.int32),
            pltpu.VMEM((CH,), jnp.int32),
            pltpu.VMEM((CH, 128), jnp.float32),
            pltpu.VMEM_SHARED((NPAD, 128), jnp.float32),
            pltpu.SemaphoreType.DMA,
        ],
    )


def _make_gin(Df, agg_mode, B):
    G = N // B

    def body(fl, fr, al, ar, w, b, sc, a_mat, h3, pt, lout, m):
        i = pl.program_id(0)
        f = jnp.concatenate([fl[0], fr[0]], axis=1)
        if agg_mode == "concat":
            agg = jnp.concatenate([al[0], ar[0]], axis=1)
        else:
            agg = al[0] + ar[0]
        t = sc[0, 0] * f + agg
        h = jnp.dot(t, w[...], preferred_element_type=jnp.float32) + b[...]
        h = _leaky(h)
        h3[0] = h[:, : H // 2]
        h3[1] = h[:, H // 2 :]
        s = jnp.dot(h, a_mat[...], preferred_element_type=jnp.float32)
        bm = jnp.max(s, axis=0, keepdims=True)

        @pl.when(i == 0)
        def _():
            p = jnp.exp(s - bm)
            m[...] = bm
            lout[...] = jnp.sum(p, axis=0, keepdims=True)
            pt[...] = lax.dot_general(
                h, p, (((0,), (0,)), ((), ())), preferred_element_type=jnp.float32
            )

        @pl.when(i > 0)
        def _():
            mn = jnp.maximum(m[...], bm)
            alpha = jnp.exp(m[...] - mn)
            p = jnp.exp(s - mn)
            m[...] = mn
            lout[...] = lout[...] * alpha + jnp.sum(p, axis=0, keepdims=True)
            pt[...] = pt[...] * alpha + lax.dot_general(
                h, p, (((0,), (0,)), ((), ())), preferred_element_type=jnp.float32
            )

    D = 2 * Df
    wa = 128 if agg_mode == "sum" else 128
    return pl.pallas_call(
        body,
        grid=(G,),
        in_specs=[
            pl.BlockSpec((1, B, Df), lambda i: (0, i, 0)),
            pl.BlockSpec((1, B, Df), lambda i: (1, i, 0)),
            pl.BlockSpec((1, B, wa), lambda i: (0, i, 0)),
            pl.BlockSpec((1, B, wa), lambda i: (1, i, 0)),
            pl.BlockSpec((D, H), lambda i: (0, 0)),
            pl.BlockSpec((1, H), lambda i: (0, 0)),
            pl.BlockSpec((1, 1), lambda i: (0, 0), memory_space=pltpu.SMEM),
            pl.BlockSpec((H, K), lambda i: (0, 0)),
        ],
        out_specs=[
            pl.BlockSpec((2, B, H // 2), lambda i: (0, i, 0)),
            pl.BlockSpec((H, K), lambda i: (0, 0)),
            pl.BlockSpec((1, K), lambda i: (0, 0)),
        ],
        out_shape=[
            jax.ShapeDtypeStruct((2, N, H // 2), jnp.float32),
            jax.ShapeDtypeStruct((H, K), jnp.float32),
            jax.ShapeDtypeStruct((1, K), jnp.float32),
        ],
        scratch_shapes=[pltpu.VMEM((1, K), jnp.float32)],
    )


def _head1_body(p0, l0, p1, l1, p2, l2, wc, bc, out):
    merged_t = jnp.concatenate(
        [p0[...] / l0[...], p1[...] / l1[...], p2[...] / l2[...]], axis=0
    )
    conv_t = _leaky(
        jnp.dot(wc[...], merged_t, preferred_element_type=jnp.float32) + bc[...]
    )
    out[...] = conv_t


_head1 = pl.pallas_call(
    _head1_body, out_shape=jax.ShapeDtypeStruct((C, K), jnp.float32)
)


def _head2_body(fc, wd1, bd1, wd2, bd2, out):
    d1 = _leaky(
        lax.dot_general(
            fc[...], wd1[...], (((1,), (1,)), ((), ())),
            preferred_element_type=jnp.float32,
        )
        + bd1[...]
    )
    o = (
        lax.dot_general(
            d1, wd2[...], (((1,), (1,)), ((), ())),
            preferred_element_type=jnp.float32,
        )
        + bd2[...]
    )
    out[...] = 1.0 / (1.0 + jnp.exp(-o))


_head2 = pl.pallas_call(
    _head2_body, out_shape=jax.ShapeDtypeStruct((1, 2), jnp.float32)
)

_SEGSUM = {}
_GIN = {}


def _segsum(feat, src2, dst2, zeros, epad, colsplit):
    key = (epad, colsplit)
    if key not in _SEGSUM:
        _SEGSUM[key] = _make_segsum(epad, colsplit)
    return _SEGSUM[key](feat, src2, dst2, zeros)


def _gin(f3, agg3, w, b, scale, a_mat, df, agg_mode):
    key = (df, agg_mode)
    if key not in _GIN:
        _GIN[key] = _make_gin(df, agg_mode, 1000)
    return _GIN[key](f3, f3, agg3, agg3, w, b, scale, a_mat)


def kernel(x, edge_index, W0, b0, eps0, A0, W1, b1, eps1, A1, W2, b2, eps2, A2,
           Wc, bc, Wd1, bd1, Wd2, bd2):
    src = edge_index[0]
    dst = edge_index[1]
    e = src.shape[0]
    step = NC * NS * CH
    epad = -(-e // step) * step
    pad = epad - e
    dst2 = jnp.concatenate([dst, jnp.full((pad,), N, jnp.int32)])
    srcp = jnp.concatenate([src, jnp.zeros((pad,), jnp.int32)])
    src2 = jnp.concatenate([srcp, srcp + N])
    zeros128 = jnp.zeros((NPAD, 128), jnp.float32)

    x3 = x.reshape(N, 2, 64).transpose(1, 0, 2)
    agg0 = _segsum(x, src2, dst2, zeros128, epad, False)
    h0, p0, l0 = _gin(x3, agg0.reshape(2, NPAD, 128), W0, b0.reshape(1, H),
                      (1.0 + eps0).reshape(1, 1), A0, 64, "sum")

    agg1 = _segsum(h0.reshape(2 * N, 128), src2, dst2, zeros128, epad, True)
    h1, p1, l1 = _gin(h0, agg1.reshape(2, NPAD, 128), W1, b1.reshape(1, H),
                      (1.0 + eps1).reshape(1, 1), A1, 128, "concat")

    agg2 = _segsum(h1.reshape(2 * N, 128), src2, dst2, zeros128, epad, True)
    h2, p2, l2 = _gin(h1, agg2.reshape(2, NPAD, 128), W2, b2.reshape(1, H),
                      (1.0 + eps2).reshape(1, 1), A2, 128, "concat")

    conv_t = _head1(p0, l0, p1, l1, p2, l2, Wc, bc.reshape(C, 1))
    fc = conv_t.reshape(1, C * K)
    return _head2(fc, Wd1, bd1.reshape(1, 128), Wd2, bd2.reshape(1, 2))
